# baseline (speedup 1.0000x reference)
import functools

import jax
import jax.numpy as jnp
from jax import lax
from jax.experimental import pallas as pl
from jax.experimental.pallas import tpu as pltpu
from jax.experimental.pallas import tpu_sc as plsc

_E = 16
_G = 4
_T = 12 * 2048
_CAP = 2048.0
_ROW_TOKENS = 1024
_ROW_WORDS = _ROW_TOKENS * _E
_BLOCKS = _ROW_TOKENS // _E

_info = plsc.get_sparse_core_info()
_NC, _NS = _info.num_cores, _info.num_subcores
_NW = _NC * _NS
_RPW = 3
_ACC = _RPW * _E * _E

_LN2 = 0.6931471805599453
_SQRT2 = 1.4142135623730951


def _vlog(s):
    bits = lax.bitcast_convert_type(s, jnp.int32)
    e = jnp.right_shift(bits, 23) - 127
    mant = jnp.bitwise_or(jnp.bitwise_and(bits, 0x007FFFFF), 0x3F800000)
    f = lax.bitcast_convert_type(mant, jnp.float32)
    big = f >= _SQRT2
    f = jnp.where(big, f * 0.5, f)
    e = e + jnp.where(big, 1, 0)
    t = (f - 1.0) / (f + 1.0)
    t2 = t * t
    w = 2.0 * t * (1.0 + t2 * (1.0 / 3.0 + t2 * (0.2 + t2 * (1.0 / 7.0 + t2 / 9.0))))
    return e.astype(jnp.float32) * _LN2 + w


def _tree(f, xs):
    while len(xs) > 1:
        nxt = [f(xs[i], xs[i + 1]) for i in range(0, len(xs) - 1, 2)]
        if len(xs) % 2:
            nxt.append(xs[-1])
        xs = nxt
    return xs[0]


_EOFF = [(e // 8) * 8192 + (e % 8) * 128 for e in range(_E)]


def _block(buf, dynbase, zvec, off, pacc, cacc):
    l = [buf[pl.ds(_EOFF[e] + dynbase, _E)] for e in range(_E)]
    m = _tree(jnp.maximum, l)
    ex = [jnp.exp(le - m) for le in l]
    s = _tree(lambda a, b: a + b, ex)
    r = 1.0 / s
    for e in range(_E):
        plsc.addupdate(pacc.at[pl.ds(off + e * _E, _E)], ex[e] * r)
        plsc.addupdate(cacc.at[pl.ds(off + e * _E, _E)],
                       jnp.where(l[e] == m, 1.0, 0.0))
    logz = m + _vlog(s)
    return zvec + logz * logz


def _row_body(buf, j, pacc, cacc, zvec):
    off = j * _E * _E

    def body(k, z):
        dynbase = (k >> 3) * 1024 + (k & 7) * _E
        return _block(buf, dynbase, z, off, pacc, cacc)

    return lax.fori_loop(0, _BLOCKS, body, zvec, unroll=4)


@functools.partial(
    pl.kernel,
    out_type=(
        jax.ShapeDtypeStruct((_NW * _ACC,), jnp.float32),
        jax.ShapeDtypeStruct((_NW * _ACC,), jnp.float32),
        jax.ShapeDtypeStruct((_NW * _E,), jnp.float32),
    ),
    mesh=plsc.VectorSubcoreMesh(core_axis_name="c", subcore_axis_name="s"),
    compiler_params=pltpu.CompilerParams(needs_layout_passes=False,
                                         disable_bounds_checks=True,
                                         disable_semaphore_checks=True,
                                         skip_device_barrier=True),
    scratch_types=[
        pltpu.VMEM((_ROW_WORDS,), jnp.float32),
        pltpu.VMEM((_ROW_WORDS,), jnp.float32),
        pltpu.VMEM((_ACC,), jnp.float32),
        pltpu.VMEM((_ACC,), jnp.float32),
        pltpu.VMEM((_E,), jnp.float32),
        pltpu.SemaphoreType.DMA,
        pltpu.SemaphoreType.DMA,
    ],
)
def _router_loss_sc(x_hbm, p_out, c_out, z_out,
                    buf0, buf1, pacc, cacc, zacc, sem0, sem1):
    wid = lax.axis_index("s") * _NC + lax.axis_index("c")
    g = wid // 8
    q = wid % 8
    zero = jnp.zeros((_E,), jnp.float32)
    for i in range(_ACC // _E):
        pacc[pl.ds(i * _E, _E)] = zero
        cacc[pl.ds(i * _E, _E)] = zero

    def half_slab(n):
        m = q * _RPW + n
        i, h = m // 2, m % 2
        slab = (i * _G + g) * (2 * _ROW_WORDS)
        lo = slab + h * 8192
        return lo

    def copy_row(n, buf, sem):
        lo = half_slab(n)
        ch = pltpu.async_copy(x_hbm.at[pl.ds(lo, 8192)],
                              buf.at[pl.ds(0, 8192)], sem)
        cl = pltpu.async_copy(x_hbm.at[pl.ds(lo + 16384, 8192)],
                              buf.at[pl.ds(8192, 8192)], sem)
        return ch, cl

    cp0 = copy_row(0, buf0, sem0)
    cp1 = copy_row(1, buf1, sem1)
    cp0[0].wait()
    cp0[1].wait()
    z0 = _row_body(buf0, 0, pacc, cacc, zero)
    cp2 = copy_row(2, buf0, sem0)
    cp1[0].wait()
    cp1[1].wait()
    z1 = _row_body(buf1, 1, pacc, cacc, z0)
    cp2[0].wait()
    cp2[1].wait()
    z2 = _row_body(buf0, 2, pacc, cacc, z1)

    zacc[...] = z2
    pltpu.sync_copy(pacc, p_out.at[pl.ds(wid * _ACC, _ACC)])
    pltpu.sync_copy(cacc, c_out.at[pl.ds(wid * _ACC, _ACC)])
    pltpu.sync_copy(zacc, z_out.at[pl.ds(wid * _E, _E)])


def kernel(router_outputs, attention_mask):
    del attention_mask
    x = (router_outputs.transpose(0, 1, 3, 2)
         .reshape(12, _G, 2, 8, 16, 128)
         .transpose(0, 1, 2, 4, 3, 5)
         .reshape(-1))
    p_parts, c_parts, z_parts = _router_loss_sc(x)
    probs = p_parts.reshape(_G, 8 * _RPW, _E, _E).sum(axis=(1, 3))
    counts = c_parts.reshape(_G, 8 * _RPW, _E, _E).sum(axis=(1, 3))
    z_loss = z_parts.sum() / (_G * _T)
    clipped = jnp.minimum(counts, _CAP)
    overflow = jnp.sum(counts - clipped, axis=1)
    final_c = clipped.at[:, 0].add(overflow)
    aux = jnp.mean((final_c / _T) * (probs / _T)) * _E ** 2
    return (0.001 * z_loss + 0.001 * aux).astype(jnp.float32)


# speedup vs baseline: 1.0789x; 1.0789x over previous
import functools

import jax
import jax.numpy as jnp
from jax import lax
from jax.experimental import pallas as pl
from jax.experimental.pallas import tpu as pltpu
from jax.experimental.pallas import tpu_sc as plsc

_E = 16
_G = 4
_T = 12 * 2048
_CAP = 2048.0
_ROW_TOKENS = 1024
_ROW_WORDS = _ROW_TOKENS * _E
_BLOCKS = _ROW_TOKENS // _E

_info = plsc.get_sparse_core_info()
_NC, _NS = _info.num_cores, _info.num_subcores
_NW = _NC * _NS
_L_SC = 8
_L_TC = 12 - _L_SC
_RPW = 2 * _L_SC // 8
_ACC = _RPW * _E * _E

_LN2 = 0.6931471805599453
_SQRT2 = 1.4142135623730951


def _vlog(s):
    bits = lax.bitcast_convert_type(s, jnp.int32)
    e = jnp.right_shift(bits, 23) - 127
    mant = jnp.bitwise_or(jnp.bitwise_and(bits, 0x007FFFFF), 0x3F800000)
    f = lax.bitcast_convert_type(mant, jnp.float32)
    big = f >= _SQRT2
    f = jnp.where(big, f * 0.5, f)
    e = e + jnp.where(big, 1, 0)
    t = (f - 1.0) / (f + 1.0)
    t2 = t * t
    w = 2.0 * t * (1.0 + t2 * (1.0 / 3.0 + t2 * (0.2 + t2 * (1.0 / 7.0 + t2 / 9.0))))
    return e.astype(jnp.float32) * _LN2 + w


def _tree(f, xs):
    while len(xs) > 1:
        nxt = [f(xs[i], xs[i + 1]) for i in range(0, len(xs) - 1, 2)]
        if len(xs) % 2:
            nxt.append(xs[-1])
        xs = nxt
    return xs[0]


_EOFF = [(e // 8) * 8192 + (e % 8) * 128 for e in range(_E)]


def _block(buf, dynbase, zvec, off, pacc, cacc):
    l = [buf[pl.ds(_EOFF[e] + dynbase, _E)] for e in range(_E)]
    m = _tree(jnp.maximum, l)
    ex = [jnp.exp(le - m) for le in l]
    s = _tree(lambda a, b: a + b, ex)
    r = 1.0 / s
    for e in range(_E):
        plsc.addupdate(pacc.at[pl.ds(off + e * _E, _E)], ex[e] * r)
        plsc.addupdate(cacc.at[pl.ds(off + e * _E, _E)],
                       jnp.where(l[e] == m, 1.0, 0.0))
    logz = m + _vlog(s)
    return zvec + logz * logz


def _row_body(buf, j, pacc, cacc, zvec):
    off = j * _E * _E

    def body(k, z):
        dynbase = (k >> 3) * 1024 + (k & 7) * _E
        return _block(buf, dynbase, z, off, pacc, cacc)

    return lax.fori_loop(0, _BLOCKS, body, zvec)


@functools.partial(
    pl.kernel,
    out_type=(
        jax.ShapeDtypeStruct((_NW * _ACC,), jnp.float32),
        jax.ShapeDtypeStruct((_NW * _ACC,), jnp.float32),
        jax.ShapeDtypeStruct((_NW * _E,), jnp.float32),
    ),
    mesh=plsc.VectorSubcoreMesh(core_axis_name="c", subcore_axis_name="s"),
    compiler_params=pltpu.CompilerParams(needs_layout_passes=False,
                                         disable_bounds_checks=True,
                                         disable_semaphore_checks=True,
                                         skip_device_barrier=True),
    scratch_types=[
        pltpu.VMEM((_ROW_WORDS,), jnp.float32),
        pltpu.VMEM((_ROW_WORDS,), jnp.float32),
        pltpu.VMEM((_ACC,), jnp.float32),
        pltpu.VMEM((_ACC,), jnp.float32),
        pltpu.VMEM((_E,), jnp.float32),
        pltpu.SemaphoreType.DMA,
        pltpu.SemaphoreType.DMA,
    ],
)
def _router_loss_sc(x_hbm, p_out, c_out, z_out,
                    buf0, buf1, pacc, cacc, zacc, sem0, sem1):
    wid = lax.axis_index("s") * _NC + lax.axis_index("c")
    g = wid // 8
    q = wid % 8
    zero = jnp.zeros((_E,), jnp.float32)
    for i in range(_ACC // _E):
        pacc[pl.ds(i * _E, _E)] = zero
        cacc[pl.ds(i * _E, _E)] = zero

    def half_slab(n):
        m = q * _RPW + n
        i, h = m // 2, m % 2
        slab = (i * _G + g) * (2 * _ROW_WORDS)
        lo = slab + h * 8192
        return lo

    def copy_row(n, buf, sem):
        lo = half_slab(n)
        ch = pltpu.async_copy(x_hbm.at[pl.ds(lo, 8192)],
                              buf.at[pl.ds(0, 8192)], sem)
        cl = pltpu.async_copy(x_hbm.at[pl.ds(lo + 16384, 8192)],
                              buf.at[pl.ds(8192, 8192)], sem)
        return ch, cl

    cp0 = copy_row(0, buf0, sem0)
    cp1 = copy_row(1, buf1, sem1)
    cp0[0].wait()
    cp0[1].wait()
    z0 = _row_body(buf0, 0, pacc, cacc, zero)
    cp1[0].wait()
    cp1[1].wait()
    z1 = _row_body(buf1, 1, pacc, cacc, z0)

    zacc[...] = z1
    pltpu.sync_copy(pacc, p_out.at[pl.ds(wid * _ACC, _ACC)])
    pltpu.sync_copy(cacc, c_out.at[pl.ds(wid * _ACC, _ACC)])
    pltpu.sync_copy(zacc, z_out.at[pl.ds(wid * _E, _E)])


def _tc_body(x_ref, p_ref, c_ref, z_ref):
    g = pl.program_id(0)
    first = jnp.logical_and(g == 0, pl.program_id(1) == 0)
    x = x_ref[0, 0]
    m = jnp.max(x, axis=0, keepdims=True)
    ex = jnp.exp(x - m)
    s = jnp.sum(ex, axis=0, keepdims=True)
    pvec = jnp.sum(ex / s, axis=1)
    ids = lax.broadcasted_iota(jnp.int32, (_E, 2048), 0)
    cand = jnp.where(x == m, ids, _E)
    idx = jnp.min(cand, axis=0, keepdims=True)
    cnt = jnp.sum(jnp.where(idx == ids, 1.0, 0.0), axis=1)
    logz = m + jnp.log(s)
    zrow = logz * logz

    @pl.when(first)
    def _():
        p_ref[...] = jnp.zeros_like(p_ref)
        c_ref[...] = jnp.zeros_like(c_ref)
        z_ref[...] = jnp.zeros_like(z_ref)

    g16 = lax.broadcasted_iota(jnp.int32, (_G, _E), 0)
    gz = lax.broadcasted_iota(jnp.int32, (_G, 2048), 0)
    p_ref[...] += jnp.where(g16 == g, pvec.reshape(1, _E), 0.0)
    c_ref[...] += jnp.where(g16 == g, cnt.reshape(1, _E), 0.0)
    z_ref[...] += jnp.where(gz == g, zrow, 0.0)


_tc_call = pl.pallas_call(
    _tc_body,
    grid=(_G, _L_TC),
    in_specs=[pl.BlockSpec((1, 1, _E, 2048), lambda g, l: (_L_SC + l, g, 0, 0))],
    out_specs=(
        pl.BlockSpec((_G, _E), lambda g, l: (0, 0)),
        pl.BlockSpec((_G, _E), lambda g, l: (0, 0)),
        pl.BlockSpec((_G, 2048), lambda g, l: (0, 0)),
    ),
    out_shape=(
        jax.ShapeDtypeStruct((_G, _E), jnp.float32),
        jax.ShapeDtypeStruct((_G, _E), jnp.float32),
        jax.ShapeDtypeStruct((_G, 2048), jnp.float32),
    ),
)


def kernel(router_outputs, attention_mask):
    del attention_mask
    x_sc = (router_outputs.transpose(0, 1, 3, 2)
            .reshape(12, _G, 2, 8, 16, 128)
            .transpose(0, 1, 2, 4, 3, 5)
            .reshape(-1))
    x_tc = router_outputs.transpose(0, 1, 3, 2)
    p_parts, c_parts, z_parts = _router_loss_sc(x_sc)
    p_tc, c_tc, z_tc = _tc_call(x_tc)
    probs = p_parts.reshape(_G, 8 * _RPW, _E, _E).sum(axis=(1, 3)) + p_tc
    counts = c_parts.reshape(_G, 8 * _RPW, _E, _E).sum(axis=(1, 3)) + c_tc
    z_loss = (z_parts.sum() + z_tc.sum()) / (_G * _T)
    clipped = jnp.minimum(counts, _CAP)
    overflow = jnp.sum(counts - clipped, axis=1)
    final_c = clipped.at[:, 0].add(overflow)
    aux = jnp.mean((final_c / _T) * (probs / _T)) * _E ** 2
    return (0.001 * z_loss + 0.001 * aux).astype(jnp.float32)


# speedup vs baseline: 1.0950x; 1.0149x over previous
import functools

import jax
import jax.numpy as jnp
from jax import lax
from jax.experimental import pallas as pl
from jax.experimental.pallas import tpu as pltpu
from jax.experimental.pallas import tpu_sc as plsc

_E = 16
_G = 4
_T = 12 * 2048
_CAP = 2048.0
_ROW_TOKENS = 1024
_ROW_WORDS = _ROW_TOKENS * _E
_BLOCKS = _ROW_TOKENS // _E

_info = plsc.get_sparse_core_info()
_NC, _NS = _info.num_cores, _info.num_subcores
_NW = _NC * _NS
_L_SC = 8
_L_TC = 12 - _L_SC
_RPW = 2 * _L_SC // 8
_ACC = _RPW * _E * _E

_LN2 = 0.6931471805599453
_SQRT2 = 1.4142135623730951


def _vlog(s):
    bits = lax.bitcast_convert_type(s, jnp.int32)
    e = jnp.right_shift(bits, 23) - 127
    mant = jnp.bitwise_or(jnp.bitwise_and(bits, 0x007FFFFF), 0x3F800000)
    f = lax.bitcast_convert_type(mant, jnp.float32)
    big = f >= _SQRT2
    f = jnp.where(big, f * 0.5, f)
    e = e + jnp.where(big, 1, 0)
    t = (f - 1.0) / (f + 1.0)
    t2 = t * t
    w = 2.0 * t * (1.0 + t2 * (1.0 / 3.0 + t2 * (0.2 + t2 * (1.0 / 7.0 + t2 / 9.0))))
    return e.astype(jnp.float32) * _LN2 + w


def _tree(f, xs):
    while len(xs) > 1:
        nxt = [f(xs[i], xs[i + 1]) for i in range(0, len(xs) - 1, 2)]
        if len(xs) % 2:
            nxt.append(xs[-1])
        xs = nxt
    return xs[0]


_EOFF = [(e // 8) * 8192 + (e % 8) * 128 for e in range(_E)]


def _block(buf, dynbase, zvec, off, pacc, cacc):
    l = [buf[pl.ds(_EOFF[e] + dynbase, _E)] for e in range(_E)]
    m = _tree(jnp.maximum, l)
    ex = [jnp.exp(le - m) for le in l]
    s = _tree(lambda a, b: a + b, ex)
    r = 1.0 / s
    for e in range(_E):
        plsc.addupdate(pacc.at[pl.ds(off + e * _E, _E)], ex[e] * r)
        plsc.addupdate(cacc.at[pl.ds(off + e * _E, _E)],
                       jnp.where(l[e] == m, 1.0, 0.0))
    logz = m + _vlog(s)
    return zvec + logz * logz


def _row_body(buf, j, pacc, cacc, zvec):
    off = j * _E * _E

    def body(k, z):
        dynbase = (k >> 3) * 1024 + (k & 7) * _E
        return _block(buf, dynbase, z, off, pacc, cacc)

    return lax.fori_loop(0, _BLOCKS, body, zvec)


@functools.partial(
    pl.kernel,
    out_type=(
        jax.ShapeDtypeStruct((_NW * _E,), jnp.float32),
        jax.ShapeDtypeStruct((_NW * _E,), jnp.float32),
        jax.ShapeDtypeStruct((_NW * _E,), jnp.float32),
    ),
    mesh=plsc.VectorSubcoreMesh(core_axis_name="c", subcore_axis_name="s"),
    compiler_params=pltpu.CompilerParams(needs_layout_passes=False,
                                         disable_bounds_checks=True,
                                         disable_semaphore_checks=True,
                                         skip_device_barrier=True),
    scratch_types=[
        pltpu.VMEM((_ROW_WORDS,), jnp.float32),
        pltpu.VMEM((_ROW_WORDS,), jnp.float32),
        pltpu.VMEM((_ACC,), jnp.float32),
        pltpu.VMEM((_ACC,), jnp.float32),
        pltpu.VMEM((_E,), jnp.float32),
        pltpu.SemaphoreType.DMA,
        pltpu.SemaphoreType.DMA,
    ],
)
def _router_loss_sc(x_hbm, p_out, c_out, z_out,
                    buf0, buf1, pacc, cacc, zacc, sem0, sem1):
    wid = lax.axis_index("s") * _NC + lax.axis_index("c")
    g = wid // 8
    q = wid % 8
    zero = jnp.zeros((_E,), jnp.float32)
    for i in range(_ACC // _E):
        pacc[pl.ds(i * _E, _E)] = zero
        cacc[pl.ds(i * _E, _E)] = zero

    def half_slab(n):
        m = q * _RPW + n
        i, h = m // 2, m % 2
        slab = (i * _G + g) * (2 * _ROW_WORDS)
        lo = slab + h * 8192
        return lo

    def copy_row(n, buf, sem):
        lo = half_slab(n)
        ch = pltpu.async_copy(x_hbm.at[pl.ds(lo, 8192)],
                              buf.at[pl.ds(0, 8192)], sem)
        cl = pltpu.async_copy(x_hbm.at[pl.ds(lo + 16384, 8192)],
                              buf.at[pl.ds(8192, 8192)], sem)
        return ch, cl

    cp0 = copy_row(0, buf0, sem0)
    cp1 = copy_row(1, buf1, sem1)
    cp0[0].wait()
    cp0[1].wait()
    z0 = _row_body(buf0, 0, pacc, cacc, zero)
    cp1[0].wait()
    cp1[1].wait()
    z1 = _row_body(buf1, 1, pacc, cacc, z0)
    zacc[...] = z1

    iota = lax.iota(jnp.int32, _E)
    iota16 = iota * _E
    for w in range(_E):
        for n in range(1, _RPW):
            off = n * _E * _E + w * _E
            pacc[pl.ds(w * _E, _E)] = (pacc[pl.ds(w * _E, _E)]
                                       + pacc[pl.ds(off, _E)])
            cacc[pl.ds(w * _E, _E)] = (cacc[pl.ds(w * _E, _E)]
                                       + cacc[pl.ds(off, _E)])
    pcols = [plsc.load_gather(pacc, [iota16 + t]) for t in range(_E)]
    ccols = [plsc.load_gather(cacc, [iota16 + t]) for t in range(_E)]
    pacc[pl.ds(0, _E)] = _tree(lambda a, b: a + b, pcols)
    cacc[pl.ds(0, _E)] = _tree(lambda a, b: a + b, ccols)
    pltpu.sync_copy(pacc.at[pl.ds(0, _E)], p_out.at[pl.ds(wid * _E, _E)])
    pltpu.sync_copy(cacc.at[pl.ds(0, _E)], c_out.at[pl.ds(wid * _E, _E)])
    pltpu.sync_copy(zacc, z_out.at[pl.ds(wid * _E, _E)])


def _tc_body(x_ref, p_ref, c_ref, z_ref):
    g = pl.program_id(0)
    first = jnp.logical_and(g == 0, pl.program_id(1) == 0)
    x = x_ref[0, 0]
    m = jnp.max(x, axis=0, keepdims=True)
    ex = jnp.exp(x - m)
    s = jnp.sum(ex, axis=0, keepdims=True)
    pvec = jnp.sum(ex / s, axis=1)
    ids = lax.broadcasted_iota(jnp.int32, (_E, 2048), 0)
    cand = jnp.where(x == m, ids, _E)
    idx = jnp.min(cand, axis=0, keepdims=True)
    cnt = jnp.sum(jnp.where(idx == ids, 1.0, 0.0), axis=1)
    logz = m + jnp.log(s)
    zrow = logz * logz

    @pl.when(first)
    def _():
        p_ref[...] = jnp.zeros_like(p_ref)
        c_ref[...] = jnp.zeros_like(c_ref)
        z_ref[...] = jnp.zeros_like(z_ref)

    g16 = lax.broadcasted_iota(jnp.int32, (_G, _E), 0)
    gz = lax.broadcasted_iota(jnp.int32, (_G, 2048), 0)
    p_ref[...] += jnp.where(g16 == g, pvec.reshape(1, _E), 0.0)
    c_ref[...] += jnp.where(g16 == g, cnt.reshape(1, _E), 0.0)
    z_ref[...] += jnp.where(gz == g, zrow, 0.0)


_tc_call = pl.pallas_call(
    _tc_body,
    grid=(_G, _L_TC),
    in_specs=[pl.BlockSpec((1, 1, _E, 2048), lambda g, l: (_L_SC + l, g, 0, 0))],
    out_specs=(
        pl.BlockSpec((_G, _E), lambda g, l: (0, 0)),
        pl.BlockSpec((_G, _E), lambda g, l: (0, 0)),
        pl.BlockSpec((_G, 2048), lambda g, l: (0, 0)),
    ),
    out_shape=(
        jax.ShapeDtypeStruct((_G, _E), jnp.float32),
        jax.ShapeDtypeStruct((_G, _E), jnp.float32),
        jax.ShapeDtypeStruct((_G, 2048), jnp.float32),
    ),
)


def kernel(router_outputs, attention_mask):
    del attention_mask
    x_sc = (router_outputs.transpose(0, 1, 3, 2)
            .reshape(12, _G, 2, 8, 16, 128)
            .transpose(0, 1, 2, 4, 3, 5)
            .reshape(-1))
    x_tc = router_outputs.transpose(0, 1, 3, 2)
    p_parts, c_parts, z_parts = _router_loss_sc(x_sc)
    p_tc, c_tc, z_tc = _tc_call(x_tc)
    probs = p_parts.reshape(_G, 8, _E).sum(axis=1) + p_tc
    counts = c_parts.reshape(_G, 8, _E).sum(axis=1) + c_tc
    z_loss = (z_parts.sum() + z_tc.sum()) / (_G * _T)
    clipped = jnp.minimum(counts, _CAP)
    overflow = jnp.sum(counts - clipped, axis=1)
    final_c = clipped.at[:, 0].add(overflow)
    aux = jnp.mean((final_c / _T) * (probs / _T)) * _E ** 2
    return (0.001 * z_loss + 0.001 * aux).astype(jnp.float32)


# speedup vs baseline: 1.3050x; 1.1918x over previous
import functools

import jax
import jax.numpy as jnp
from jax import lax
from jax.experimental import pallas as pl
from jax.experimental.pallas import tpu as pltpu
from jax.experimental.pallas import tpu_sc as plsc

_E = 16
_G = 4
_T = 12 * 2048
_CAP = 2048.0
_ROW_TOKENS = 1024
_ROW_WORDS = _ROW_TOKENS * _E
_BLOCKS = _ROW_TOKENS // _E

_info = plsc.get_sparse_core_info()
_NC, _NS = _info.num_cores, _info.num_subcores
_NW = _NC * _NS
_L_SC = 8
_L_TC = 12 - _L_SC
_RPW = 2 * _L_SC // 8
_ACC = _RPW * _E * _E

_LN2 = 0.6931471805599453
_SQRT2 = 1.4142135623730951


def _vlog(s):
    bits = lax.bitcast_convert_type(s, jnp.int32)
    e = jnp.right_shift(bits, 23) - 127
    mant = jnp.bitwise_or(jnp.bitwise_and(bits, 0x007FFFFF), 0x3F800000)
    f = lax.bitcast_convert_type(mant, jnp.float32)
    big = f >= _SQRT2
    f = jnp.where(big, f * 0.5, f)
    e = e + jnp.where(big, 1, 0)
    t = (f - 1.0) / (f + 1.0)
    t2 = t * t
    w = 2.0 * t * (1.0 + t2 * (1.0 / 3.0 + t2 * (0.2 + t2 * (1.0 / 7.0 + t2 / 9.0))))
    return e.astype(jnp.float32) * _LN2 + w


def _tree(f, xs):
    while len(xs) > 1:
        nxt = [f(xs[i], xs[i + 1]) for i in range(0, len(xs) - 1, 2)]
        if len(xs) % 2:
            nxt.append(xs[-1])
        xs = nxt
    return xs[0]


_EOFF = [(e // 8) * 8192 + (e % 8) * 128 for e in range(_E)]


def _block(buf, dynbase, zvec, off, pacc, cacc):
    l = [buf[pl.ds(_EOFF[e] + dynbase, _E)] for e in range(_E)]
    m = _tree(jnp.maximum, l)
    ex = [jnp.exp(le - m) for le in l]
    s = _tree(lambda a, b: a + b, ex)
    r = 1.0 / s
    for e in range(_E):
        plsc.addupdate(pacc.at[pl.ds(off + e * _E, _E)], ex[e] * r)
        plsc.addupdate(cacc.at[pl.ds(off + e * _E, _E)],
                       jnp.where(l[e] == m, 1.0, 0.0))
    logz = m + _vlog(s)
    return zvec + logz * logz


def _row_body(buf, j, pacc, cacc, zvec):
    off = j * _E * _E

    def body(k, z):
        dynbase = (k >> 3) * 1024 + (k & 7) * _E
        return _block(buf, dynbase, z, off, pacc, cacc)

    return lax.fori_loop(0, _BLOCKS, body, zvec)


@functools.partial(
    pl.kernel,
    out_type=(
        jax.ShapeDtypeStruct((_NW * _E,), jnp.float32),
        jax.ShapeDtypeStruct((_NW * _E,), jnp.float32),
        jax.ShapeDtypeStruct((_NW * _E,), jnp.float32),
    ),
    mesh=plsc.VectorSubcoreMesh(core_axis_name="c", subcore_axis_name="s"),
    compiler_params=pltpu.CompilerParams(needs_layout_passes=False,
                                         disable_bounds_checks=True,
                                         disable_semaphore_checks=True,
                                         skip_device_barrier=True),
    scratch_types=[
        pltpu.VMEM((_ROW_WORDS,), jnp.float32),
        pltpu.VMEM((_ROW_WORDS,), jnp.float32),
        pltpu.VMEM((_ACC,), jnp.float32),
        pltpu.VMEM((_ACC,), jnp.float32),
        pltpu.VMEM((_E,), jnp.float32),
        pltpu.SemaphoreType.DMA,
        pltpu.SemaphoreType.DMA,
    ],
)
def _router_loss_sc(x_hbm, p_out, c_out, z_out,
                    buf0, buf1, pacc, cacc, zacc, sem0, sem1):
    wid = lax.axis_index("s") * _NC + lax.axis_index("c")
    g = wid // 8
    q = wid % 8
    zero = jnp.zeros((_E,), jnp.float32)
    for i in range(_ACC // _E):
        pacc[pl.ds(i * _E, _E)] = zero
        cacc[pl.ds(i * _E, _E)] = zero

    def half_slab(n):
        m = q * _RPW + n
        i, h = m // 2, m % 2
        slab = (i * _G + g) * (2 * _ROW_WORDS)
        lo = slab + h * 8192
        return lo

    def copy_row(n, buf, sem):
        lo = half_slab(n)
        ch = pltpu.async_copy(x_hbm.at[pl.ds(lo, 8192)],
                              buf.at[pl.ds(0, 8192)], sem)
        cl = pltpu.async_copy(x_hbm.at[pl.ds(lo + 16384, 8192)],
                              buf.at[pl.ds(8192, 8192)], sem)
        return ch, cl

    cp0 = copy_row(0, buf0, sem0)
    cp1 = copy_row(1, buf1, sem1)
    cp0[0].wait()
    cp0[1].wait()
    z0 = _row_body(buf0, 0, pacc, cacc, zero)
    cp1[0].wait()
    cp1[1].wait()
    z1 = _row_body(buf1, 1, pacc, cacc, z0)
    zacc[...] = z1

    iota = lax.iota(jnp.int32, _E)
    iota16 = iota * _E
    for w in range(_E):
        for n in range(1, _RPW):
            off = n * _E * _E + w * _E
            pacc[pl.ds(w * _E, _E)] = (pacc[pl.ds(w * _E, _E)]
                                       + pacc[pl.ds(off, _E)])
            cacc[pl.ds(w * _E, _E)] = (cacc[pl.ds(w * _E, _E)]
                                       + cacc[pl.ds(off, _E)])
    pcols = [plsc.load_gather(pacc, [iota16 + t]) for t in range(_E)]
    ccols = [plsc.load_gather(cacc, [iota16 + t]) for t in range(_E)]
    pacc[pl.ds(0, _E)] = _tree(lambda a, b: a + b, pcols)
    cacc[pl.ds(0, _E)] = _tree(lambda a, b: a + b, ccols)
    pltpu.sync_copy(pacc.at[pl.ds(0, _E)], p_out.at[pl.ds(wid * _E, _E)])
    pltpu.sync_copy(cacc.at[pl.ds(0, _E)], c_out.at[pl.ds(wid * _E, _E)])
    pltpu.sync_copy(zacc, z_out.at[pl.ds(wid * _E, _E)])


def _tc_body(x_ref, p_ref, c_ref, z_ref):
    g = pl.program_id(0)
    first = jnp.logical_and(g == 0, pl.program_id(1) == 0)
    x = x_ref[0, 0]
    m = jnp.max(x, axis=0, keepdims=True)
    ex = jnp.exp(x - m)
    s = jnp.sum(ex, axis=0, keepdims=True)
    pvec = jnp.sum(ex / s, axis=1)
    ids = lax.broadcasted_iota(jnp.int32, (_E, 2048), 0)
    cand = jnp.where(x == m, ids, _E)
    idx = jnp.min(cand, axis=0, keepdims=True)
    cnt = jnp.sum(jnp.where(idx == ids, 1.0, 0.0), axis=1)
    logz = m + jnp.log(s)
    zrow = logz * logz

    @pl.when(first)
    def _():
        p_ref[...] = jnp.zeros_like(p_ref)
        c_ref[...] = jnp.zeros_like(c_ref)
        z_ref[...] = jnp.zeros_like(z_ref)

    g16 = lax.broadcasted_iota(jnp.int32, (_G, _E), 0)
    gz = lax.broadcasted_iota(jnp.int32, (_G, 2048), 0)
    p_ref[...] += jnp.where(g16 == g, pvec.reshape(1, _E), 0.0)
    c_ref[...] += jnp.where(g16 == g, cnt.reshape(1, _E), 0.0)
    z_ref[...] += jnp.where(gz == g, zrow, 0.0)


_tc_call = pl.pallas_call(
    _tc_body,
    grid=(_G, _L_TC),
    in_specs=[pl.BlockSpec((1, 1, _E, 2048), lambda g, l: (_L_SC + l, g, 0, 0))],
    out_specs=(
        pl.BlockSpec((_G, _E), lambda g, l: (0, 0)),
        pl.BlockSpec((_G, _E), lambda g, l: (0, 0)),
        pl.BlockSpec((_G, 2048), lambda g, l: (0, 0)),
    ),
    out_shape=(
        jax.ShapeDtypeStruct((_G, _E), jnp.float32),
        jax.ShapeDtypeStruct((_G, _E), jnp.float32),
        jax.ShapeDtypeStruct((_G, 2048), jnp.float32),
    ),
)


def _combine_body(p_sc_ref, c_sc_ref, z_sc_ref, p_tc_ref, c_tc_ref, z_tc_ref,
                  out_ref):
    def fold(ref):
        x = ref[...]
        return _tree(lambda a, b: a + b,
                     [x[:, q * _E:(q + 1) * _E] for q in range(8)])

    probs = fold(p_sc_ref) + p_tc_ref[...]
    counts = fold(c_sc_ref) + c_tc_ref[...]
    z_total = jnp.sum(z_sc_ref[...]) + jnp.sum(z_tc_ref[...])
    z_loss = z_total / (_G * _T)
    clipped = jnp.minimum(counts, _CAP)
    overflow = jnp.sum(counts - clipped, axis=1, keepdims=True)
    lane = lax.broadcasted_iota(jnp.int32, (_G, _E), 1)
    final_c = jnp.where(lane == 0, clipped + overflow, clipped)
    aux = jnp.mean((final_c / _T) * (probs / _T)) * _E ** 2
    out_ref[...] = jnp.full((1, 1), 0.001 * z_loss + 0.001 * aux, jnp.float32)


_combine_call = pl.pallas_call(
    _combine_body,
    out_shape=jax.ShapeDtypeStruct((1, 1), jnp.float32),
)


def kernel(router_outputs, attention_mask):
    del attention_mask
    x_sc = (router_outputs.transpose(0, 1, 3, 2)
            .reshape(12, _G, 2, 8, 16, 128)
            .transpose(0, 1, 2, 4, 3, 5)
            .reshape(-1))
    x_tc = router_outputs.transpose(0, 1, 3, 2)
    p_parts, c_parts, z_parts = _router_loss_sc(x_sc)
    p_tc, c_tc, z_tc = _tc_call(x_tc)
    out = _combine_call(p_parts.reshape(_G, 8 * _E),
                        c_parts.reshape(_G, 8 * _E),
                        z_parts.reshape(_G, 8 * _E),
                        p_tc, c_tc, z_tc)
    return out.reshape(())


# speedup vs baseline: 1.4833x; 1.1366x over previous
import functools

import jax
import jax.numpy as jnp
from jax import lax
from jax.experimental import pallas as pl
from jax.experimental.pallas import tpu as pltpu
from jax.experimental.pallas import tpu_sc as plsc

_E = 16
_G = 4
_T = 12 * 2048
_CAP = 2048.0
_ROW_TOKENS = 1024
_ROW_WORDS = _ROW_TOKENS * _E
_BLOCKS = _ROW_TOKENS // _E

_info = plsc.get_sparse_core_info()
_NC, _NS = _info.num_cores, _info.num_subcores
_NW = _NC * _NS
_L_SC = 8
_L_TC = 12 - _L_SC
_RPW = 2 * _L_SC // 8
_ACC = _RPW * _E * _E

_LN2 = 0.6931471805599453
_SQRT2 = 1.4142135623730951


def _vlog(s):
    bits = lax.bitcast_convert_type(s, jnp.int32)
    e = jnp.right_shift(bits, 23) - 127
    mant = jnp.bitwise_or(jnp.bitwise_and(bits, 0x007FFFFF), 0x3F800000)
    f = lax.bitcast_convert_type(mant, jnp.float32)
    big = f >= _SQRT2
    f = jnp.where(big, f * 0.5, f)
    e = e + jnp.where(big, 1, 0)
    t = (f - 1.0) / (f + 1.0)
    t2 = t * t
    w = 2.0 * t * (1.0 + t2 * (1.0 / 3.0 + t2 * (0.2 + t2 * (1.0 / 7.0 + t2 / 9.0))))
    return e.astype(jnp.float32) * _LN2 + w


def _tree(f, xs):
    while len(xs) > 1:
        nxt = [f(xs[i], xs[i + 1]) for i in range(0, len(xs) - 1, 2)]
        if len(xs) % 2:
            nxt.append(xs[-1])
        xs = nxt
    return xs[0]


_EOFF = [(e // 8) * 8192 + (e % 8) * 128 for e in range(_E)]


def _block(buf, dynbase, zvec, pregs, off, cacc):
    l = [buf[pl.ds(_EOFF[e] + dynbase, _E)] for e in range(_E)]
    m = _tree(jnp.maximum, l)
    ex = [jnp.exp(le - m) for le in l]
    s = _tree(lambda a, b: a + b, ex)
    r = 1.0 / s
    pregs = [pregs[e] + ex[e] * r for e in range(_E)]
    for e in range(_E):
        plsc.addupdate(cacc.at[pl.ds(off + e * _E, _E)],
                       jnp.where(l[e] == m, 1.0, 0.0))
    logz = m + _vlog(s)
    return zvec + logz * logz, pregs


def _row_body(buf, j, cacc, zvec, pregs):
    off = j * _E * _E

    def body(k, carry):
        dynbase = (k >> 3) * 1024 + (k & 7) * _E
        z, pr = carry
        z, pr = _block(buf, dynbase, z, list(pr), off, cacc)
        return (z, tuple(pr))

    out = lax.fori_loop(0, _BLOCKS, body, (zvec, tuple(pregs)))
    return out[0], list(out[1])


@functools.partial(
    pl.kernel,
    out_type=(
        jax.ShapeDtypeStruct((_NW * _E,), jnp.float32),
        jax.ShapeDtypeStruct((_NW * _E,), jnp.float32),
        jax.ShapeDtypeStruct((_NW * _E,), jnp.float32),
    ),
    mesh=plsc.VectorSubcoreMesh(core_axis_name="c", subcore_axis_name="s"),
    compiler_params=pltpu.CompilerParams(needs_layout_passes=False,
                                         disable_bounds_checks=True,
                                         disable_semaphore_checks=True,
                                         skip_device_barrier=True),
    scratch_types=[
        pltpu.VMEM((_ROW_WORDS,), jnp.float32),
        pltpu.VMEM((_ROW_WORDS,), jnp.float32),
        pltpu.VMEM((_ACC,), jnp.float32),
        pltpu.VMEM((_ACC,), jnp.float32),
        pltpu.VMEM((_E,), jnp.float32),
        pltpu.SemaphoreType.DMA,
        pltpu.SemaphoreType.DMA,
    ],
)
def _router_loss_sc(x_hbm, p_out, c_out, z_out,
                    buf0, buf1, pacc, cacc, zacc, sem0, sem1):
    wid = lax.axis_index("s") * _NC + lax.axis_index("c")
    g = wid // 8
    q = wid % 8
    zero = jnp.zeros((_E,), jnp.float32)
    for i in range(_ACC // _E):
        cacc[pl.ds(i * _E, _E)] = zero

    def half_slab(n):
        m = q * _RPW + n
        i, h = m // 2, m % 2
        slab = (i * _G + g) * (2 * _ROW_WORDS)
        lo = slab + h * 8192
        return lo

    def copy_row(n, buf, sem):
        lo = half_slab(n)
        ch = pltpu.async_copy(x_hbm.at[pl.ds(lo, 8192)],
                              buf.at[pl.ds(0, 8192)], sem)
        cl = pltpu.async_copy(x_hbm.at[pl.ds(lo + 16384, 8192)],
                              buf.at[pl.ds(8192, 8192)], sem)
        return ch, cl

    cp0 = copy_row(0, buf0, sem0)
    cp1 = copy_row(1, buf1, sem1)
    pregs = [zero] * _E
    cp0[0].wait()
    cp0[1].wait()
    z0, pregs = _row_body(buf0, 0, cacc, zero, pregs)
    cp1[0].wait()
    cp1[1].wait()
    z1, pregs = _row_body(buf1, 1, cacc, z0, pregs)
    zacc[...] = z1

    iota = lax.iota(jnp.int32, _E)
    iota16 = iota * _E
    for w in range(_E):
        pacc[pl.ds(w * _E, _E)] = pregs[w]
        for n in range(1, _RPW):
            off = n * _E * _E + w * _E
            cacc[pl.ds(w * _E, _E)] = (cacc[pl.ds(w * _E, _E)]
                                       + cacc[pl.ds(off, _E)])
    pcols = [plsc.load_gather(pacc, [iota16 + t]) for t in range(_E)]
    ccols = [plsc.load_gather(cacc, [iota16 + t]) for t in range(_E)]
    pacc[pl.ds(0, _E)] = _tree(lambda a, b: a + b, pcols)
    cacc[pl.ds(0, _E)] = _tree(lambda a, b: a + b, ccols)
    pltpu.sync_copy(pacc.at[pl.ds(0, _E)], p_out.at[pl.ds(wid * _E, _E)])
    pltpu.sync_copy(cacc.at[pl.ds(0, _E)], c_out.at[pl.ds(wid * _E, _E)])
    pltpu.sync_copy(zacc, z_out.at[pl.ds(wid * _E, _E)])


def _tc_body(x_ref, p_ref, c_ref, z_ref):
    g = pl.program_id(0)
    first = jnp.logical_and(g == 0, pl.program_id(1) == 0)
    x = x_ref[0, 0]
    m = jnp.max(x, axis=0, keepdims=True)
    ex = jnp.exp(x - m)
    s = jnp.sum(ex, axis=0, keepdims=True)
    pvec = jnp.sum(ex / s, axis=1)
    ids = lax.broadcasted_iota(jnp.int32, (_E, 2048), 0)
    cand = jnp.where(x == m, ids, _E)
    idx = jnp.min(cand, axis=0, keepdims=True)
    cnt = jnp.sum(jnp.where(idx == ids, 1.0, 0.0), axis=1)
    logz = m + jnp.log(s)
    zrow = logz * logz

    @pl.when(first)
    def _():
        p_ref[...] = jnp.zeros_like(p_ref)
        c_ref[...] = jnp.zeros_like(c_ref)
        z_ref[...] = jnp.zeros_like(z_ref)

    g16 = lax.broadcasted_iota(jnp.int32, (_G, _E), 0)
    gz = lax.broadcasted_iota(jnp.int32, (_G, 2048), 0)
    p_ref[...] += jnp.where(g16 == g, pvec.reshape(1, _E), 0.0)
    c_ref[...] += jnp.where(g16 == g, cnt.reshape(1, _E), 0.0)
    z_ref[...] += jnp.where(gz == g, zrow, 0.0)


_tc_call = pl.pallas_call(
    _tc_body,
    grid=(_G, _L_TC),
    in_specs=[pl.BlockSpec((1, 1, _E, 2048), lambda g, l: (_L_SC + l, g, 0, 0))],
    out_specs=(
        pl.BlockSpec((_G, _E), lambda g, l: (0, 0)),
        pl.BlockSpec((_G, _E), lambda g, l: (0, 0)),
        pl.BlockSpec((_G, 2048), lambda g, l: (0, 0)),
    ),
    out_shape=(
        jax.ShapeDtypeStruct((_G, _E), jnp.float32),
        jax.ShapeDtypeStruct((_G, _E), jnp.float32),
        jax.ShapeDtypeStruct((_G, 2048), jnp.float32),
    ),
)


def _combine_body(p_sc_ref, c_sc_ref, z_sc_ref, p_tc_ref, c_tc_ref, z_tc_ref,
                  out_ref):
    def fold(ref):
        x = ref[...]
        return _tree(lambda a, b: a + b,
                     [x[:, q * _E:(q + 1) * _E] for q in range(8)])

    probs = fold(p_sc_ref) + p_tc_ref[...]
    counts = fold(c_sc_ref) + c_tc_ref[...]
    z_total = jnp.sum(z_sc_ref[...]) + jnp.sum(z_tc_ref[...])
    z_loss = z_total / (_G * _T)
    clipped = jnp.minimum(counts, _CAP)
    overflow = jnp.sum(counts - clipped, axis=1, keepdims=True)
    lane = lax.broadcasted_iota(jnp.int32, (_G, _E), 1)
    final_c = jnp.where(lane == 0, clipped + overflow, clipped)
    aux = jnp.mean((final_c / _T) * (probs / _T)) * _E ** 2
    out_ref[...] = jnp.full((1, 1), 0.001 * z_loss + 0.001 * aux, jnp.float32)


_combine_call = pl.pallas_call(
    _combine_body,
    out_shape=jax.ShapeDtypeStruct((1, 1), jnp.float32),
)


def kernel(router_outputs, attention_mask):
    del attention_mask
    x_sc = (router_outputs.transpose(0, 1, 3, 2)
            .reshape(12, _G, 2, 8, 16, 128)
            .transpose(0, 1, 2, 4, 3, 5)
            .reshape(-1))
    x_tc = router_outputs.transpose(0, 1, 3, 2)
    p_parts, c_parts, z_parts = _router_loss_sc(x_sc)
    p_tc, c_tc, z_tc = _tc_call(x_tc)
    out = _combine_call(p_parts.reshape(_G, 8 * _E),
                        c_parts.reshape(_G, 8 * _E),
                        z_parts.reshape(_G, 8 * _E),
                        p_tc, c_tc, z_tc)
    return out.reshape(())
